# baseline (speedup 1.0000x reference)
import functools

import jax
import jax.numpy as jnp
from jax import lax
from jax.experimental import pallas as pl
from jax.experimental.pallas import tpu as pltpu
from jax.experimental.pallas import tpu_sc as plsc

N = 10000
NPAD = 10240
D = 128
E = 160000
NC = 2
NS = 16
NW = NC * NS
NTILE = NPAD // NS
CHUNK = 128
NCHUNK = E // CHUNK
KMAX = -(-NCHUNK // NW)

def _wid(c, s):
    return s * NC + c


NB = 4


def _sc_degree_body(ep, et, eg, out, ix0, ix1, ix2, ix3, ones_v, stage_v,
                    acc_p, acc_t, acc_g, isems, ssems):
    c = lax.axis_index("c")
    s = lax.axis_index("s")
    w = _wid(c, s)
    ixs = (ix0, ix1, ix2, ix3)
    one16 = jnp.ones((16,), jnp.float32)
    zero16 = jnp.zeros((16,), jnp.float32)
    for j in range(CHUNK // 16):
        ones_v[pl.ds(j * 16, 16)] = one16
    for j in range(NTILE // 16):
        stage_v[pl.ds(j * 16, 16)] = zero16
    for acc in (acc_p, acc_t, acc_g):
        pltpu.sync_copy(stage_v, acc.at[pl.ds(s * NTILE, NTILE)])
    plsc.subcore_barrier()

    cols = (ep, et, eg)
    accs = (acc_p, acc_t, acc_g)
    TOT = 3 * KMAX

    def cid_of(t):
        return w + NW * (t % KMAX)

    def start_idx(t):
        if not (0 <= t < TOT):
            return

        @pl.when(cid_of(t) < NCHUNK)
        def _():
            pltpu.async_copy(
                cols[t // KMAX].at[pl.ds(E + cid_of(t) * CHUNK, CHUNK)],
                ixs[t % NB], isems.at[t % NB])

    def wait_idx(t):
        @pl.when(cid_of(t) < NCHUNK)
        def _():
            pltpu.make_async_copy(
                cols[t // KMAX].at[pl.ds(0, CHUNK)],
                ixs[t % NB], isems.at[t % NB]).wait()

    def start_scatter(t):
        @pl.when(cid_of(t) < NCHUNK)
        def _():
            pltpu.async_copy(ones_v, accs[t // KMAX].at[ixs[t % NB]],
                             ssems.at[t % NB])

    def wait_scatter(t):
        if not (0 <= t < TOT):
            return

        @pl.when(cid_of(t) < NCHUNK)
        def _():
            pltpu.make_async_copy(
                ones_v, accs[t // KMAX].at[ixs[t % NB]],
                ssems.at[t % NB]).wait()

    for t in range(TOT):
        @pl.when(cid_of(t) < NCHUNK)
        def _(t=t):
            pltpu.sync_copy(
                cols[t // KMAX].at[pl.ds(E + cid_of(t) * CHUNK, CHUNK)],
                ixs[0])
            pltpu.sync_copy(ones_v, accs[t // KMAX].at[ixs[0]], add=True)
    plsc.subcore_barrier()

    for g, acc in enumerate(accs):
        pltpu.sync_copy(acc.at[pl.ds(s * NTILE, NTILE)], stage_v)
        pltpu.sync_copy(
            stage_v, out.at[pl.ds((c * 3 + g) * NPAD + s * NTILE, NTILE)])


def _sc_lgconv_body(table, edges, out, ibuf, rb0, rb1,
                    acc_sh, cx0, cx1, cx2, cx3, isems, gsems, ssems, wsem):
    c = lax.axis_index("c")
    s = lax.axis_index("s")
    w = _wid(c, s)
    cxs = (cx0, cx1, cx2, cx3)
    rbufs = (rb0, rb1)
    zero16 = jnp.zeros((16,), jnp.float32)
    for r in range(CHUNK):
        for j in range(D // 16):
            rb0[r, pl.ds(j * 16, 16)] = zero16
    zds = []
    for z in range(NTILE // CHUNK):
        zds.append(pltpu.async_copy(
            rb0, acc_sh.at[pl.ds(s * NTILE + z * CHUNK, CHUNK)], wsem))
    for d in zds:
        d.wait()
    plsc.subcore_barrier()

    def cid_of(k):
        return w + NW * k

    def guarded(k, fn):
        if not (0 <= k < KMAX):
            return

        @pl.when(cid_of(k) < NCHUNK)
        def _():
            fn(k)

    def start_idx(k):
        base = cid_of(k) * CHUNK
        pltpu.async_copy(edges.at[pl.ds(base, CHUNK)],
                         ibuf.at[k % NB], isems.at[k % NB])
        pltpu.async_copy(edges.at[pl.ds(E + base, CHUNK)],
                         cxs[k % NB], isems.at[k % NB])

    def wait_idx(k):
        pltpu.make_async_copy(edges.at[pl.ds(0, CHUNK)],
                              ibuf.at[k % NB], isems.at[k % NB]).wait()
        pltpu.make_async_copy(edges.at[pl.ds(0, CHUNK)],
                              cxs[k % NB], isems.at[k % NB]).wait()

    def start_gather(k):
        pltpu.async_copy(table.at[ibuf.at[k % NB]], rbufs[k % 2],
                         gsems.at[k % 2])

    def wait_gather(k):
        pltpu.make_async_copy(table.at[pl.ds(0, CHUNK)],
                              rbufs[k % 2], gsems.at[k % 2]).wait()

    def start_scatter(k):
        pltpu.async_copy(rbufs[k % 2], acc_sh.at[cxs[k % NB]],
                         ssems.at[k % 2], add=True)

    def wait_scatter(k):
        pltpu.make_async_copy(rbufs[k % 2], acc_sh.at[cxs[k % NB]],
                              ssems.at[k % 2]).wait()

    guarded(0, start_idx)
    guarded(1, start_idx)
    guarded(0, wait_idx)
    guarded(0, start_gather)
    for k in range(KMAX):
        guarded(k - 1, wait_scatter)
        guarded(k + 1, wait_idx)
        guarded(k + 1, start_gather)
        guarded(k + 2, start_idx)
        guarded(k, wait_gather)
        guarded(k, start_scatter)
    guarded(KMAX - 1, wait_scatter)
    plsc.subcore_barrier()

    wds = []
    for z in range(NTILE // CHUNK):
        off = s * NTILE + z * CHUNK
        b = z % 2
        if z >= 2:
            wds[z - 2].wait()
        pltpu.sync_copy(acc_sh.at[pl.ds(off, CHUNK)], rbufs[b])
        wds.append(pltpu.async_copy(rbufs[b], out.at[c, pl.ds(off, CHUNK)],
                                    gsems.at[b]))
    for d in wds[-2:]:
        d.wait()


def _sc_final_body(tp, tt, tg, idxf, fe_all, out,
                   ibuf, gbuf, fev, isems, gsems, osems, fsem):
    c = lax.axis_index("c")
    s = lax.axis_index("s")
    w = _wid(c, s)
    tabs = (tp, tt, tg)
    TOT = 2 * KMAX

    def parts(m):
        side = m // KMAX
        cid = w + NW * (m % KMAX)
        ok = (cid < NCHUNK) & (m < TOT)
        if isinstance(m, int):
            ok = ok & (0 <= m < TOT)
        else:
            ok = ok & (m >= 0)
        return side, cid, ok

    def start_idx(m, p):
        side, cid, ok = parts(m)

        @pl.when(ok)
        def _():
            for t in range(3):
                pltpu.async_copy(
                    idxf.at[pl.ds((side * 3 + t) * E + cid * CHUNK, CHUNK)],
                    ibuf.at[p, t], isems.at[p])

    def wait_idx(m, p):
        _, _, ok = parts(m)

        @pl.when(ok)
        def _():
            for t in range(3):
                pltpu.make_async_copy(idxf.at[pl.ds(0, CHUNK)],
                                      ibuf.at[p, t], isems.at[p]).wait()

    def start_gathers(m, p):
        _, _, ok = parts(m)

        @pl.when(ok)
        def _():
            for t in range(3):
                pltpu.async_copy(tabs[t].at[ibuf.at[p, t]],
                                 gbuf.at[p, t], gsems.at[p])

    def wait_gathers(m, p):
        _, _, ok = parts(m)

        @pl.when(ok)
        def _():
            for t in range(3):
                pltpu.make_async_copy(tabs[t].at[pl.ds(0, CHUNK)],
                                      gbuf.at[p, t], gsems.at[p]).wait()

    def start_fe(m):
        side, cid, ok = parts(m)

        @pl.when(ok)
        def _():
            pltpu.async_copy(fe_all.at[side, pl.ds(cid * CHUNK, CHUNK)],
                             fev, fsem)

    def wait_fe(m):
        _, _, ok = parts(m)

        @pl.when(ok)
        def _():
            pltpu.make_async_copy(fe_all.at[0, pl.ds(0, CHUNK)],
                                  fev, fsem).wait()

    def compute(m, p):
        _, _, ok = parts(m)

        @pl.when(ok)
        def _():
            def row(r4, _):
                for u in range(4):
                    r = r4 * 4 + u
                    for j in range(D // 16):
                        sl = pl.ds(j * 16, 16)
                        gbuf[p, 0, r, sl] = (
                            (gbuf[p, 0, r, sl] + gbuf[p, 1, r, sl])
                            + (gbuf[p, 2, r, sl] + fev[r, sl]))
                return 0

            lax.fori_loop(0, CHUNK // 4, row, 0)

    def start_out(m, p):
        side, cid, ok = parts(m)

        @pl.when(ok)
        def _():
            pltpu.async_copy(gbuf.at[p, 0],
                             out.at[side, pl.ds(cid * CHUNK, CHUNK)],
                             osems.at[p])

    def wait_out(m, p):
        side, cid, ok = parts(m)

        @pl.when(ok)
        def _():
            pltpu.make_async_copy(gbuf.at[p, 0],
                                  out.at[side, pl.ds(cid * CHUNK, CHUNK)],
                                  osems.at[p]).wait()

    start_idx(0, 0)
    start_idx(1, 1)
    wait_idx(0, 0)
    start_gathers(0, 0)
    start_fe(0)

    def pair(i, _):
        for j in range(2):
            m = 2 * i + j
            p = j
            wait_gathers(m, p)
            wait_fe(m)
            start_idx(m + 2, p)
            wait_idx(m + 1, 1 - p)
            wait_out(m - 1, 1 - p)
            start_gathers(m + 1, 1 - p)
            compute(m, p)
            start_out(m, p)
            start_fe(m + 1)
        return 0

    lax.fori_loop(0, TOT // 2, pair, 0)
    wait_out(TOT - 1, (TOT - 1) % 2)


@functools.cache
def _sc_kernels():
    mesh = plsc.VectorSubcoreMesh(
        core_axis_name="c", subcore_axis_name="s",
        num_cores=NC, num_subcores=NS)
    sc_degree = pl.kernel(
        _sc_degree_body,
        out_type=jax.ShapeDtypeStruct((NC * 3 * NPAD,), jnp.float32),
        mesh=mesh,
        scratch_types=[
            pltpu.VMEM((CHUNK,), jnp.int32),
            pltpu.VMEM((CHUNK,), jnp.int32),
            pltpu.VMEM((CHUNK,), jnp.int32),
            pltpu.VMEM((CHUNK,), jnp.int32),
            pltpu.VMEM((CHUNK,), jnp.float32),
            pltpu.VMEM((NTILE,), jnp.float32),
            pltpu.VMEM_SHARED((NPAD,), jnp.float32),
            pltpu.VMEM_SHARED((NPAD,), jnp.float32),
            pltpu.VMEM_SHARED((NPAD,), jnp.float32),
            pltpu.SemaphoreType.DMA((NB,)),
            pltpu.SemaphoreType.DMA((NB,)),
        ],
    )
    sc_lgconv = pl.kernel(
        _sc_lgconv_body,
        out_type=jax.ShapeDtypeStruct((NC, NPAD, D), jnp.float32),
        mesh=mesh,
        scratch_types=[
            pltpu.VMEM((NB, CHUNK), jnp.int32),
            pltpu.VMEM((CHUNK, D), jnp.float32),
            pltpu.VMEM((CHUNK, D), jnp.float32),
            pltpu.VMEM_SHARED((NPAD, D), jnp.float32),
            pltpu.VMEM((CHUNK,), jnp.int32),
            pltpu.VMEM((CHUNK,), jnp.int32),
            pltpu.VMEM((CHUNK,), jnp.int32),
            pltpu.VMEM((CHUNK,), jnp.int32),
            pltpu.SemaphoreType.DMA((NB,)),
            pltpu.SemaphoreType.DMA((2,)),
            pltpu.SemaphoreType.DMA((2,)),
            pltpu.SemaphoreType.DMA,
        ],
    )
    sc_final = pl.kernel(
        _sc_final_body,
        out_type=jax.ShapeDtypeStruct((2, E, D), jnp.float32),
        mesh=mesh,
        scratch_types=[
            pltpu.VMEM((2, 3, CHUNK), jnp.int32),
            pltpu.VMEM((2, 3, CHUNK, D), jnp.float32),
            pltpu.VMEM((CHUNK, D), jnp.float32),
            pltpu.SemaphoreType.DMA((2,)),
            pltpu.SemaphoreType.DMA((2,)),
            pltpu.SemaphoreType.DMA((2,)),
            pltpu.SemaphoreType.DMA,
        ],
    )
    return sc_degree, sc_lgconv, sc_final


RB = 2048
EB = 2000


def _dinv_from(deg_block):
    deg = deg_block[0] + deg_block[1]
    return jnp.where(deg > 0.0,
                     lax.rsqrt(jnp.maximum(deg, 1e-12)),
                     0.0)


def _stage_a_body(degp, x0p, x0t, x0g, ap, at_, ag):
    dinv = _dinv_from(degp[...])
    ap[...] = x0p[...] * dinv[0][:, None]
    at_[...] = x0t[...] * dinv[1][:, None]
    ag[...] = x0g[...] * dinv[2][:, None]


def _stage_b_body(degp, s1p, s1t, s1g, bp, bt, bg):
    dinv = _dinv_from(degp[...])
    for g, (s1, b) in enumerate(((s1p, bp), (s1t, bt), (s1g, bg))):
        d2 = (dinv[g] * dinv[g])[:, None]
        b[...] = (s1[0] + s1[1]) * d2


def _stage_c_body(degp, x0p, x0t, x0g, s1p, s1t, s1g, s2p, s2t, s2g,
                  op_, ot_, og_):
    dinv = _dinv_from(degp[...])
    third = jnp.float32(1.0 / 9.0)
    for g, (x0, s1, s2, o) in enumerate(
            ((x0p, s1p, s2p, op_), (x0t, s1t, s2t, ot_),
             (x0g, s1g, s2g, og_))):
        d = dinv[g][:, None]
        conv = (s1[0] + s1[1]) + (s2[0] + s2[1])
        o[...] = (x0[...] + d * conv) * third


def _matmul_body(feats, ws, bs, fe_out):
    r = jnp.dot(feats[0].astype(jnp.bfloat16), ws[0].astype(jnp.bfloat16),
                preferred_element_type=jnp.float32) + bs[0]
    fe_out[...] = r[None]


def _node_grid_specs(n_tab):
    deg_spec = pl.BlockSpec((NC, 3, RB), lambda i: (0, 0, i))
    tab_spec = pl.BlockSpec((RB, D), lambda i: (i, 0))
    part_spec = pl.BlockSpec((NC, RB, D), lambda i: (0, i, 0))
    return deg_spec, tab_spec, part_spec


def _tc_stage_a(degp, x0p, x0t, x0g):
    deg_spec, tab_spec, _ = _node_grid_specs(3)
    out = jax.ShapeDtypeStruct((NPAD, D), jnp.float32)
    return pl.pallas_call(
        _stage_a_body,
        grid=(NPAD // RB,),
        in_specs=[deg_spec, tab_spec, tab_spec, tab_spec],
        out_specs=[tab_spec, tab_spec, tab_spec],
        out_shape=[out, out, out],
    )(degp, x0p, x0t, x0g)


def _tc_stage_b(degp, s1p, s1t, s1g):
    deg_spec, tab_spec, part_spec = _node_grid_specs(3)
    out = jax.ShapeDtypeStruct((NPAD, D), jnp.float32)
    return pl.pallas_call(
        _stage_b_body,
        grid=(NPAD // RB,),
        in_specs=[deg_spec, part_spec, part_spec, part_spec],
        out_specs=[tab_spec, tab_spec, tab_spec],
        out_shape=[out, out, out],
    )(degp, s1p, s1t, s1g)


def _tc_stage_c(degp, x0p, x0t, x0g, s1p, s1t, s1g, s2p, s2t, s2g):
    deg_spec, tab_spec, part_spec = _node_grid_specs(3)
    out = jax.ShapeDtypeStruct((NPAD, D), jnp.float32)
    return pl.pallas_call(
        _stage_c_body,
        grid=(NPAD // RB,),
        in_specs=[deg_spec, tab_spec, tab_spec, tab_spec,
                  part_spec, part_spec, part_spec,
                  part_spec, part_spec, part_spec],
        out_specs=[tab_spec, tab_spec, tab_spec],
        out_shape=[out, out, out],
    )(degp, x0p, x0t, x0g, s1p, s1t, s1g, s2p, s2t, s2g)


def _tc_matmul(feats, ws, bs):
    feat_spec = pl.BlockSpec((1, EB, 16), lambda s, i: (s, i, 0))
    w_spec = pl.BlockSpec((1, 16, D), lambda s, i: (s, 0, 0))
    b_spec = pl.BlockSpec((1, 1, D), lambda s, i: (s, 0, 0))
    out_spec = pl.BlockSpec((1, EB, D), lambda s, i: (s, i, 0))
    return pl.pallas_call(
        _matmul_body,
        grid=(2, E // EB),
        in_specs=[feat_spec, w_spec, b_spec],
        out_specs=out_spec,
        out_shape=jax.ShapeDtypeStruct((2, E, D), jnp.float32),
    )(feats, ws, bs)


def kernel(user_problem_edge_index, user_test_edge_index, user_tag_edge_index,
           out_src_feature_engineering, out_dst_feature_engineering,
           W_problem, W_test, W_tag, W_src_fe, b_src_fe, W_dst_fe, b_dst_fe):
    pe = user_problem_edge_index.astype(jnp.int32)
    te = user_test_edge_index.astype(jnp.int32)
    ge = user_tag_edge_index.astype(jnp.int32)
    pef, tef, gef = pe.reshape(-1), te.reshape(-1), ge.reshape(-1)
    idxf = jnp.concatenate([pe[0], te[0], ge[0], pe[1], te[1], ge[1]])

    def pad(w):
        return jnp.zeros((NPAD, D), jnp.float32).at[:N].set(w)

    x0p, x0t, x0g = pad(W_problem), pad(W_test), pad(W_tag)

    _sc_degree, _sc_lgconv, _sc_final = _sc_kernels()
    degp = _sc_degree(pef, tef, gef).reshape(NC, 3, NPAD)
    ap, at_, ag = _tc_stage_a(degp, x0p, x0t, x0g)
    s1p = _sc_lgconv(ap, pef)
    s1t = _sc_lgconv(at_, tef)
    s1g = _sc_lgconv(ag, gef)
    bp, bt, bg = _tc_stage_b(degp, s1p, s1t, s1g)
    s2p = _sc_lgconv(bp, pef)
    s2t = _sc_lgconv(bt, tef)
    s2g = _sc_lgconv(bg, gef)
    op_, ot_, og_ = _tc_stage_c(degp, x0p, x0t, x0g,
                                s1p, s1t, s1g, s2p, s2t, s2g)
    fe_all = _tc_matmul(
        jnp.stack([out_src_feature_engineering, out_dst_feature_engineering]),
        jnp.stack([W_src_fe, W_dst_fe]),
        jnp.stack([b_src_fe, b_dst_fe]).reshape(2, 1, D))
    return _sc_final(op_, ot_, og_, idxf, fe_all)


# speedup vs baseline: 1.1243x; 1.1243x over previous
import functools

import jax
import jax.numpy as jnp
from jax import lax
from jax.experimental import pallas as pl
from jax.experimental.pallas import tpu as pltpu
from jax.experimental.pallas import tpu_sc as plsc

N = 10000
NPAD = 10240
D = 128
E = 160000
NC = 2
NS = 16
NW = NC * NS
NTILE = NPAD // NS
CHUNK = 128
NCHUNK = E // CHUNK
KMAX = -(-NCHUNK // NW)

def _wid(c, s):
    return s * NC + c


NB = 4


def _sc_degree_body(ep, et, eg, out, ix0, ix1, ix2, ix3, ones_v, stage_v,
                    acc_p, acc_t, acc_g, isems, ssems):
    c = lax.axis_index("c")
    s = lax.axis_index("s")
    w = _wid(c, s)
    ixs = (ix0, ix1, ix2, ix3)
    one16 = jnp.ones((16,), jnp.float32)
    zero16 = jnp.zeros((16,), jnp.float32)
    for j in range(CHUNK // 16):
        ones_v[pl.ds(j * 16, 16)] = one16
    for j in range(NTILE // 16):
        stage_v[pl.ds(j * 16, 16)] = zero16
    for acc in (acc_p, acc_t, acc_g):
        pltpu.sync_copy(stage_v, acc.at[pl.ds(s * NTILE, NTILE)])
    plsc.subcore_barrier()

    cols = (ep, et, eg)
    accs = (acc_p, acc_t, acc_g)
    TOT = 3 * KMAX

    def cid_of(t):
        return w + NW * (t % KMAX)

    def start_idx(t):
        if not (0 <= t < TOT):
            return

        @pl.when(cid_of(t) < NCHUNK)
        def _():
            pltpu.async_copy(
                cols[t // KMAX].at[pl.ds(E + cid_of(t) * CHUNK, CHUNK)],
                ixs[t % NB], isems.at[t % NB])

    def wait_idx(t):
        @pl.when(cid_of(t) < NCHUNK)
        def _():
            pltpu.make_async_copy(
                cols[t // KMAX].at[pl.ds(0, CHUNK)],
                ixs[t % NB], isems.at[t % NB]).wait()

    def start_scatter(t):
        @pl.when(cid_of(t) < NCHUNK)
        def _():
            pltpu.async_copy(ones_v, accs[t // KMAX].at[ixs[t % NB]],
                             ssems.at[t % NB])

    def wait_scatter(t):
        if not (0 <= t < TOT):
            return

        @pl.when(cid_of(t) < NCHUNK)
        def _():
            pltpu.make_async_copy(
                ones_v, accs[t // KMAX].at[ixs[t % NB]],
                ssems.at[t % NB]).wait()

    for t in range(TOT):
        @pl.when(cid_of(t) < NCHUNK)
        def _(t=t):
            pltpu.sync_copy(
                cols[t // KMAX].at[pl.ds(E + cid_of(t) * CHUNK, CHUNK)],
                ixs[0])
            pltpu.sync_copy(ones_v, accs[t // KMAX].at[ixs[0]], add=True)
    plsc.subcore_barrier()

    for g, acc in enumerate(accs):
        pltpu.sync_copy(acc.at[pl.ds(s * NTILE, NTILE)], stage_v)
        pltpu.sync_copy(
            stage_v, out.at[pl.ds((c * 3 + g) * NPAD + s * NTILE, NTILE)])


def _sc_lgconv_body(table, edges, out, ibuf, rb0, rb1,
                    acc_sh, cx0, cx1, cx2, cx3, isems, gsems, ssems, wsem):
    c = lax.axis_index("c")
    s = lax.axis_index("s")
    w = _wid(c, s)
    cxs = (cx0, cx1, cx2, cx3)
    rbufs = (rb0, rb1)
    zero16 = jnp.zeros((16,), jnp.float32)
    for r in range(CHUNK):
        for j in range(D // 16):
            rb0[r, pl.ds(j * 16, 16)] = zero16
    zds = []
    for z in range(NTILE // CHUNK):
        zds.append(pltpu.async_copy(
            rb0, acc_sh.at[pl.ds(s * NTILE + z * CHUNK, CHUNK)], wsem))
    for d in zds:
        d.wait()
    plsc.subcore_barrier()

    def cid_of(k):
        return w + NW * k

    def guarded(k, fn):
        if not (0 <= k < KMAX):
            return

        @pl.when(cid_of(k) < NCHUNK)
        def _():
            fn(k)

    def start_idx(k):
        base = cid_of(k) * CHUNK
        pltpu.async_copy(edges.at[pl.ds(base, CHUNK)],
                         ibuf.at[k % NB], isems.at[k % NB])
        pltpu.async_copy(edges.at[pl.ds(E + base, CHUNK)],
                         cxs[k % NB], isems.at[k % NB])

    def wait_idx(k):
        pltpu.make_async_copy(edges.at[pl.ds(0, CHUNK)],
                              ibuf.at[k % NB], isems.at[k % NB]).wait()
        pltpu.make_async_copy(edges.at[pl.ds(0, CHUNK)],
                              cxs[k % NB], isems.at[k % NB]).wait()

    def start_gather(k):
        pltpu.async_copy(table.at[ibuf.at[k % NB]], rbufs[k % 2],
                         gsems.at[k % 2])

    def wait_gather(k):
        pltpu.make_async_copy(table.at[pl.ds(0, CHUNK)],
                              rbufs[k % 2], gsems.at[k % 2]).wait()

    def start_scatter(k):
        pltpu.async_copy(rbufs[k % 2], acc_sh.at[cxs[k % NB]],
                         ssems.at[k % 2], add=True)

    def wait_scatter(k):
        pltpu.make_async_copy(rbufs[k % 2], acc_sh.at[cxs[k % NB]],
                              ssems.at[k % 2]).wait()

    guarded(0, start_idx)
    guarded(1, start_idx)
    guarded(0, wait_idx)
    guarded(0, start_gather)
    for k in range(KMAX):
        guarded(k - 1, wait_scatter)
        guarded(k + 1, wait_idx)
        guarded(k + 1, start_gather)
        guarded(k + 2, start_idx)
        guarded(k, wait_gather)
        guarded(k, start_scatter)
    guarded(KMAX - 1, wait_scatter)
    plsc.subcore_barrier()

    wds = []
    for z in range(NTILE // CHUNK):
        off = s * NTILE + z * CHUNK
        b = z % 2
        if z >= 2:
            wds[z - 2].wait()
        pltpu.sync_copy(acc_sh.at[pl.ds(off, CHUNK)], rbufs[b])
        wds.append(pltpu.async_copy(rbufs[b], out.at[c, pl.ds(off, CHUNK)],
                                    gsems.at[b]))
    for d in wds[-2:]:
        d.wait()


def _sc_final_body(tp, tt, tg, idxf, fe_all, out,
                   ibuf, gbuf, fev, isems, gsems, osems, fsem):
    c = lax.axis_index("c")
    s = lax.axis_index("s")
    w = _wid(c, s)
    tabs = (tp, tt, tg)
    TOT = 2 * KMAX

    def parts(m):
        side = m // KMAX
        cid = w + NW * (m % KMAX)
        ok = (cid < NCHUNK) & (m < TOT)
        if isinstance(m, int):
            ok = ok & (0 <= m < TOT)
        else:
            ok = ok & (m >= 0)
        return side, cid, ok

    def start_idx(m, p):
        side, cid, ok = parts(m)

        @pl.when(ok)
        def _():
            for t in range(3):
                pltpu.async_copy(
                    idxf.at[pl.ds((side * 3 + t) * E + cid * CHUNK, CHUNK)],
                    ibuf.at[p, t], isems.at[p])

    def wait_idx(m, p):
        _, _, ok = parts(m)

        @pl.when(ok)
        def _():
            for t in range(3):
                pltpu.make_async_copy(idxf.at[pl.ds(0, CHUNK)],
                                      ibuf.at[p, t], isems.at[p]).wait()

    def start_gathers(m, p):
        _, _, ok = parts(m)

        @pl.when(ok)
        def _():
            for t in range(3):
                pltpu.async_copy(tabs[t].at[ibuf.at[p, t]],
                                 gbuf.at[p, t], gsems.at[p])

    def wait_gathers(m, p):
        _, _, ok = parts(m)

        @pl.when(ok)
        def _():
            for t in range(3):
                pltpu.make_async_copy(tabs[t].at[pl.ds(0, CHUNK)],
                                      gbuf.at[p, t], gsems.at[p]).wait()

    def start_fe(m):
        side, cid, ok = parts(m)

        @pl.when(ok)
        def _():
            pltpu.async_copy(fe_all.at[side, pl.ds(cid * CHUNK, CHUNK)],
                             fev, fsem)

    def wait_fe(m):
        _, _, ok = parts(m)

        @pl.when(ok)
        def _():
            pltpu.make_async_copy(fe_all.at[0, pl.ds(0, CHUNK)],
                                  fev, fsem).wait()

    def compute(m, p):
        _, _, ok = parts(m)

        @pl.when(ok)
        def _():
            @plsc.parallel_loop(0, CHUNK, unroll=2)
            def row(r):
                for j in range(D // 16):
                    sl = pl.ds(j * 16, 16)
                    gbuf[p, 0, r, sl] = (
                        (gbuf[p, 0, r, sl] + gbuf[p, 1, r, sl])
                        + (gbuf[p, 2, r, sl] + fev[r, sl]))

    def start_out(m, p):
        side, cid, ok = parts(m)

        @pl.when(ok)
        def _():
            pltpu.async_copy(gbuf.at[p, 0],
                             out.at[side, pl.ds(cid * CHUNK, CHUNK)],
                             osems.at[p])

    def wait_out(m, p):
        side, cid, ok = parts(m)

        @pl.when(ok)
        def _():
            pltpu.make_async_copy(gbuf.at[p, 0],
                                  out.at[side, pl.ds(cid * CHUNK, CHUNK)],
                                  osems.at[p]).wait()

    start_idx(0, 0)
    start_idx(1, 1)
    wait_idx(0, 0)
    start_gathers(0, 0)
    start_fe(0)

    def pair(i, _):
        for j in range(2):
            m = 2 * i + j
            p = j
            wait_gathers(m, p)
            wait_fe(m)
            start_idx(m + 2, p)
            wait_idx(m + 1, 1 - p)
            wait_out(m - 1, 1 - p)
            start_gathers(m + 1, 1 - p)
            compute(m, p)
            start_out(m, p)
            start_fe(m + 1)
        return 0

    lax.fori_loop(0, TOT // 2, pair, 0)
    wait_out(TOT - 1, (TOT - 1) % 2)


@functools.cache
def _sc_kernels():
    mesh = plsc.VectorSubcoreMesh(
        core_axis_name="c", subcore_axis_name="s",
        num_cores=NC, num_subcores=NS)
    sc_degree = pl.kernel(
        _sc_degree_body,
        out_type=jax.ShapeDtypeStruct((NC * 3 * NPAD,), jnp.float32),
        mesh=mesh,
        scratch_types=[
            pltpu.VMEM((CHUNK,), jnp.int32),
            pltpu.VMEM((CHUNK,), jnp.int32),
            pltpu.VMEM((CHUNK,), jnp.int32),
            pltpu.VMEM((CHUNK,), jnp.int32),
            pltpu.VMEM((CHUNK,), jnp.float32),
            pltpu.VMEM((NTILE,), jnp.float32),
            pltpu.VMEM_SHARED((NPAD,), jnp.float32),
            pltpu.VMEM_SHARED((NPAD,), jnp.float32),
            pltpu.VMEM_SHARED((NPAD,), jnp.float32),
            pltpu.SemaphoreType.DMA((NB,)),
            pltpu.SemaphoreType.DMA((NB,)),
        ],
    )
    sc_lgconv = pl.kernel(
        _sc_lgconv_body,
        out_type=jax.ShapeDtypeStruct((NC, NPAD, D), jnp.float32),
        mesh=mesh,
        scratch_types=[
            pltpu.VMEM((NB, CHUNK), jnp.int32),
            pltpu.VMEM((CHUNK, D), jnp.float32),
            pltpu.VMEM((CHUNK, D), jnp.float32),
            pltpu.VMEM_SHARED((NPAD, D), jnp.float32),
            pltpu.VMEM((CHUNK,), jnp.int32),
            pltpu.VMEM((CHUNK,), jnp.int32),
            pltpu.VMEM((CHUNK,), jnp.int32),
            pltpu.VMEM((CHUNK,), jnp.int32),
            pltpu.SemaphoreType.DMA((NB,)),
            pltpu.SemaphoreType.DMA((2,)),
            pltpu.SemaphoreType.DMA((2,)),
            pltpu.SemaphoreType.DMA,
        ],
    )
    sc_final = pl.kernel(
        _sc_final_body,
        out_type=jax.ShapeDtypeStruct((2, E, D), jnp.float32),
        mesh=mesh,
        scratch_types=[
            pltpu.VMEM((2, 3, CHUNK), jnp.int32),
            pltpu.VMEM((2, 3, CHUNK, D), jnp.float32),
            pltpu.VMEM((CHUNK, D), jnp.float32),
            pltpu.SemaphoreType.DMA((2,)),
            pltpu.SemaphoreType.DMA((2,)),
            pltpu.SemaphoreType.DMA((2,)),
            pltpu.SemaphoreType.DMA,
        ],
    )
    return sc_degree, sc_lgconv, sc_final


RB = 2048
EB = 2000


def _dinv_from(deg_block):
    deg = deg_block[0] + deg_block[1]
    return jnp.where(deg > 0.0,
                     lax.rsqrt(jnp.maximum(deg, 1e-12)),
                     0.0)


def _stage_a_body(degp, x0p, x0t, x0g, ap, at_, ag):
    dinv = _dinv_from(degp[...])
    ap[...] = x0p[...] * dinv[0][:, None]
    at_[...] = x0t[...] * dinv[1][:, None]
    ag[...] = x0g[...] * dinv[2][:, None]


def _stage_b_body(degp, s1p, s1t, s1g, bp, bt, bg):
    dinv = _dinv_from(degp[...])
    for g, (s1, b) in enumerate(((s1p, bp), (s1t, bt), (s1g, bg))):
        d2 = (dinv[g] * dinv[g])[:, None]
        b[...] = (s1[0] + s1[1]) * d2


def _stage_c_body(degp, x0p, x0t, x0g, s1p, s1t, s1g, s2p, s2t, s2g,
                  op_, ot_, og_):
    dinv = _dinv_from(degp[...])
    third = jnp.float32(1.0 / 9.0)
    for g, (x0, s1, s2, o) in enumerate(
            ((x0p, s1p, s2p, op_), (x0t, s1t, s2t, ot_),
             (x0g, s1g, s2g, og_))):
        d = dinv[g][:, None]
        conv = (s1[0] + s1[1]) + (s2[0] + s2[1])
        o[...] = (x0[...] + d * conv) * third


def _matmul_body(feats, ws, bs, fe_out):
    r = jnp.dot(feats[0].astype(jnp.bfloat16), ws[0].astype(jnp.bfloat16),
                preferred_element_type=jnp.float32) + bs[0]
    fe_out[...] = r[None]


def _node_grid_specs(n_tab):
    deg_spec = pl.BlockSpec((NC, 3, RB), lambda i: (0, 0, i))
    tab_spec = pl.BlockSpec((RB, D), lambda i: (i, 0))
    part_spec = pl.BlockSpec((NC, RB, D), lambda i: (0, i, 0))
    return deg_spec, tab_spec, part_spec


def _tc_stage_a(degp, x0p, x0t, x0g):
    deg_spec, tab_spec, _ = _node_grid_specs(3)
    out = jax.ShapeDtypeStruct((NPAD, D), jnp.float32)
    return pl.pallas_call(
        _stage_a_body,
        grid=(NPAD // RB,),
        in_specs=[deg_spec, tab_spec, tab_spec, tab_spec],
        out_specs=[tab_spec, tab_spec, tab_spec],
        out_shape=[out, out, out],
    )(degp, x0p, x0t, x0g)


def _tc_stage_b(degp, s1p, s1t, s1g):
    deg_spec, tab_spec, part_spec = _node_grid_specs(3)
    out = jax.ShapeDtypeStruct((NPAD, D), jnp.float32)
    return pl.pallas_call(
        _stage_b_body,
        grid=(NPAD // RB,),
        in_specs=[deg_spec, part_spec, part_spec, part_spec],
        out_specs=[tab_spec, tab_spec, tab_spec],
        out_shape=[out, out, out],
    )(degp, s1p, s1t, s1g)


def _tc_stage_c(degp, x0p, x0t, x0g, s1p, s1t, s1g, s2p, s2t, s2g):
    deg_spec, tab_spec, part_spec = _node_grid_specs(3)
    out = jax.ShapeDtypeStruct((NPAD, D), jnp.float32)
    return pl.pallas_call(
        _stage_c_body,
        grid=(NPAD // RB,),
        in_specs=[deg_spec, tab_spec, tab_spec, tab_spec,
                  part_spec, part_spec, part_spec,
                  part_spec, part_spec, part_spec],
        out_specs=[tab_spec, tab_spec, tab_spec],
        out_shape=[out, out, out],
    )(degp, x0p, x0t, x0g, s1p, s1t, s1g, s2p, s2t, s2g)


def _tc_matmul(feats, ws, bs):
    feat_spec = pl.BlockSpec((1, EB, 16), lambda s, i: (s, i, 0))
    w_spec = pl.BlockSpec((1, 16, D), lambda s, i: (s, 0, 0))
    b_spec = pl.BlockSpec((1, 1, D), lambda s, i: (s, 0, 0))
    out_spec = pl.BlockSpec((1, EB, D), lambda s, i: (s, i, 0))
    return pl.pallas_call(
        _matmul_body,
        grid=(2, E // EB),
        in_specs=[feat_spec, w_spec, b_spec],
        out_specs=out_spec,
        out_shape=jax.ShapeDtypeStruct((2, E, D), jnp.float32),
    )(feats, ws, bs)


def kernel(user_problem_edge_index, user_test_edge_index, user_tag_edge_index,
           out_src_feature_engineering, out_dst_feature_engineering,
           W_problem, W_test, W_tag, W_src_fe, b_src_fe, W_dst_fe, b_dst_fe):
    pe = user_problem_edge_index.astype(jnp.int32)
    te = user_test_edge_index.astype(jnp.int32)
    ge = user_tag_edge_index.astype(jnp.int32)
    pef, tef, gef = pe.reshape(-1), te.reshape(-1), ge.reshape(-1)
    idxf = jnp.concatenate([pe[0], te[0], ge[0], pe[1], te[1], ge[1]])

    def pad(w):
        return jnp.zeros((NPAD, D), jnp.float32).at[:N].set(w)

    x0p, x0t, x0g = pad(W_problem), pad(W_test), pad(W_tag)

    _sc_degree, _sc_lgconv, _sc_final = _sc_kernels()
    degp = _sc_degree(pef, tef, gef).reshape(NC, 3, NPAD)
    ap, at_, ag = _tc_stage_a(degp, x0p, x0t, x0g)
    s1p = _sc_lgconv(ap, pef)
    s1t = _sc_lgconv(at_, tef)
    s1g = _sc_lgconv(ag, gef)
    bp, bt, bg = _tc_stage_b(degp, s1p, s1t, s1g)
    s2p = _sc_lgconv(bp, pef)
    s2t = _sc_lgconv(bt, tef)
    s2g = _sc_lgconv(bg, gef)
    op_, ot_, og_ = _tc_stage_c(degp, x0p, x0t, x0g,
                                s1p, s1t, s1g, s2p, s2t, s2g)
    fe_all = _tc_matmul(
        jnp.stack([out_src_feature_engineering, out_dst_feature_engineering]),
        jnp.stack([W_src_fe, W_dst_fe]),
        jnp.stack([b_src_fe, b_dst_fe]).reshape(2, 1, D))
    return _sc_final(op_, ot_, og_, idxf, fe_all)
